# baseline (speedup 1.0000x reference)
import functools

import jax
import jax.numpy as jnp
from jax.experimental import pallas as pl
from jax.experimental.pallas import tpu as pltpu
from jax.experimental.pallas import tpu_sc as plsc

_B = 512
_F = 16
_W = 16
_S = 640


def _repack(x):
    E = x.shape[0]
    nch = E // _S
    mesh = plsc.VectorSubcoreMesh(core_axis_name="c", subcore_axis_name="s")

    @functools.partial(
        pl.kernel, mesh=mesh,
        out_type=jax.ShapeDtypeStruct((E // 8, 128), jnp.float32),
        scratch_types=[pltpu.VMEM((_S, _F), jnp.float32),
                       pltpu.VMEM((_S // 8, 128), jnp.float32)],
    )
    def k(x_hbm, out_hbm, buf, buf128):
        c = jax.lax.axis_index("c")
        s = jax.lax.axis_index("s")
        wid = s * 2 + c
        n_mine = (nch - wid + 31) // 32

        def body(j, carry):
            t = wid + j * 32
            base = pl.multiple_of(t * _S, _S)
            obase = pl.multiple_of(t * (_S // 8), _S // 8)
            pltpu.sync_copy(x_hbm.at[pl.ds(base, _S)], buf)

            def row(r, c2):
                for c8 in range(8):
                    buf128[r, pl.ds(c8 * _F, _F)] = buf[r * 8 + c8, :]
                return c2

            jax.lax.fori_loop(0, _S // 8, row, 0, unroll=4)
            pltpu.sync_copy(buf128, out_hbm.at[pl.ds(obase, _S // 8)])
            return carry

        jax.lax.fori_loop(0, n_mine, body, 0)

    return k(x)


def _pool_body(starts_ref, ends_ref, seg_ref, xp_ref, w1_ref, b1_ref,
               w2rep_ref, b2_ref, out_ref, acc_ref):
    i = pl.program_id(0)

    @pl.when(i == 0)
    def _init():
        acc_ref[...] = jnp.zeros_like(acc_ref)

    xp = xp_ref[...]
    x = jnp.concatenate(
        [xp[:, _F * c:_F * (c + 1)] for c in range(8)], axis=0)
    h = jnp.tanh(
        jnp.dot(x, w1_ref[...], preferred_element_type=jnp.float32)
        + b1_ref[...])
    s = (jnp.dot(h, w2rep_ref[...], preferred_element_type=jnp.float32)
         + b2_ref[...])
    ex = jnp.exp(s)
    y = x * ex

    seg = seg_ref[0, 0, :]
    lo = starts_ref[i]
    lo_al = (lo // 8) * 8
    hi = ends_ref[i]
    nwin = (hi - lo_al) // _W + 1

    def _win(j):
        base = lo_al + j * _W
        ids = jax.lax.broadcasted_iota(jnp.int32, (_W, 1), 0) + base
        oh = (ids == seg[None, :]).astype(jnp.float32)
        num = jnp.dot(oh, y, preferred_element_type=jnp.float32)
        den = jnp.dot(oh, ex, preferred_element_type=jnp.float32)
        cur_n = acc_ref[pl.ds(base, _W), :_F]
        acc_ref[pl.ds(base, _W), :_F] = cur_n + num
        cur_d = acc_ref[pl.ds(base, _W), _F:]
        acc_ref[pl.ds(base, _W), _F:] = cur_d + den

    _win(0)

    @pl.when(nwin > 1)
    def _rest():
        jax.lax.fori_loop(1, nwin, lambda j, c: (_win(j), c)[1], 0)

    @pl.when(i == pl.num_programs(0) - 1)
    def _fin():
        acc = acc_ref[:_B, :]
        den = acc[:, _F:_F + 1]
        den = jnp.where(den == 0.0, 1.0, den)
        out_ref[...] = acc[:, :_F] / den


def kernel(edge_attr, batch_e, W1, b1, W2, b2):
    E, F = edge_attr.shape
    H = W1.shape[1]
    K = 8000 if E % 8000 == 0 else 8
    Kp = K // 8
    nblk = E // K

    xp = _repack(edge_attr.astype(jnp.float32))

    seg = batch_e.astype(jnp.int32)
    seg3 = seg.reshape(nblk, Kp, 8).transpose(0, 2, 1).reshape(nblk, 1, K)
    starts = seg[::K]
    ends = seg[K - 1::K]
    b1r = b1.reshape(1, H).astype(jnp.float32)
    w2rep = jnp.tile(W2.astype(jnp.float32), (1, _F))
    b2r = b2.reshape(1, 1).astype(jnp.float32)

    grid_spec = pltpu.PrefetchScalarGridSpec(
        num_scalar_prefetch=2,
        grid=(nblk,),
        in_specs=[
            pl.BlockSpec((1, 1, K), lambda i, *_: (i, 0, 0)),
            pl.BlockSpec((Kp, 128), lambda i, *_: (i, 0)),
            pl.BlockSpec((F, H), lambda i, *_: (0, 0)),
            pl.BlockSpec((1, H), lambda i, *_: (0, 0)),
            pl.BlockSpec((H, _F), lambda i, *_: (0, 0)),
            pl.BlockSpec((1, 1), lambda i, *_: (0, 0)),
        ],
        out_specs=pl.BlockSpec((_B, _F), lambda i, *_: (0, 0)),
        scratch_shapes=[pltpu.VMEM((_B + _W, 2 * _F), jnp.float32)],
    )
    return pl.pallas_call(
        _pool_body,
        grid_spec=grid_spec,
        out_shape=jax.ShapeDtypeStruct((_B, _F), jnp.float32),
        compiler_params=pltpu.CompilerParams(
            dimension_semantics=("arbitrary",)),
    )(starts, ends, seg3, xp, W1.astype(jnp.float32), b1r, w2rep, b2r)


# speedup vs baseline: 1.6750x; 1.6750x over previous
import functools

import jax
import jax.numpy as jnp
from jax.experimental import pallas as pl
from jax.experimental.pallas import tpu as pltpu

_B = 512
_F = 16
_W = 16


def _pool_body(starts_ref, ends_ref, seg_ref, x_ref, w1_ref, b1_ref,
               w2rep_ref, b2_ref, out_ref, acc_ref):
    i = pl.program_id(0)

    @pl.when(i == 0)
    def _init():
        acc_ref[...] = jnp.zeros_like(acc_ref)

    x = x_ref[...]
    h = jnp.tanh(
        jnp.dot(x, w1_ref[...], preferred_element_type=jnp.float32)
        + b1_ref[...])
    s = (jnp.dot(h, w2rep_ref[...], preferred_element_type=jnp.float32)
         + b2_ref[...])
    ex = jnp.exp(s)
    y = x * ex

    seg = seg_ref[0, 0, :]
    lo = starts_ref[i]
    lo_al = (lo // 8) * 8
    hi = ends_ref[i]
    nwin = (hi - lo_al) // _W + 1

    def _win(j):
        base = lo_al + j * _W
        ids = jax.lax.broadcasted_iota(jnp.int32, (_W, 1), 0) + base
        oh = (ids == seg[None, :]).astype(jnp.float32)
        num = jnp.dot(oh, y, preferred_element_type=jnp.float32)
        den = jnp.dot(oh, ex, preferred_element_type=jnp.float32)
        cur_n = acc_ref[pl.ds(base, _W), :_F]
        acc_ref[pl.ds(base, _W), :_F] = cur_n + num
        cur_d = acc_ref[pl.ds(base, _W), _F:]
        acc_ref[pl.ds(base, _W), _F:] = cur_d + den

    _win(0)

    @pl.when(nwin > 1)
    def _rest():
        jax.lax.fori_loop(1, nwin, lambda j, c: (_win(j), c)[1], 0)

    @pl.when(i == pl.num_programs(0) - 1)
    def _fin():
        acc = acc_ref[:_B, :]
        den = acc[:, _F:_F + 1]
        den = jnp.where(den == 0.0, 1.0, den)
        out_ref[...] = acc[:, :_F] / den


def kernel(edge_attr, batch_e, W1, b1, W2, b2):
    E, F = edge_attr.shape
    H = W1.shape[1]
    K = 8000 if E % 8000 == 0 else 8
    nblk = E // K

    seg = batch_e.astype(jnp.int32)
    seg3 = seg.reshape(nblk, 1, K)
    starts = seg[::K]
    ends = seg[K - 1::K]
    b1r = b1.reshape(1, H).astype(jnp.float32)
    w2rep = jnp.tile(W2.astype(jnp.float32), (1, _F))
    b2r = b2.reshape(1, 1).astype(jnp.float32)

    grid_spec = pltpu.PrefetchScalarGridSpec(
        num_scalar_prefetch=2,
        grid=(nblk,),
        in_specs=[
            pl.BlockSpec((1, 1, K), lambda i, *_: (i, 0, 0)),
            pl.BlockSpec((K, F), lambda i, *_: (i, 0)),
            pl.BlockSpec((F, H), lambda i, *_: (0, 0)),
            pl.BlockSpec((1, H), lambda i, *_: (0, 0)),
            pl.BlockSpec((H, _F), lambda i, *_: (0, 0)),
            pl.BlockSpec((1, 1), lambda i, *_: (0, 0)),
        ],
        out_specs=pl.BlockSpec((_B, _F), lambda i, *_: (0, 0)),
        scratch_shapes=[pltpu.VMEM((_B + _W, 2 * _F), jnp.float32)],
    )
    return pl.pallas_call(
        _pool_body,
        grid_spec=grid_spec,
        out_shape=jax.ShapeDtypeStruct((_B, _F), jnp.float32),
        compiler_params=pltpu.CompilerParams(
            dimension_semantics=("arbitrary",)),
    )(starts, ends, seg3, edge_attr.astype(jnp.float32),
      W1.astype(jnp.float32), b1r, w2rep, b2r)
